# baseline (speedup 1.0000x reference)
import jax
import jax.numpy as jnp
from jax.experimental import pallas as pl
from jax.experimental.pallas import tpu as pltpu

_ALPHA = 1.702
_LIMIT = 7.0
_TOP_K = 8
_B, _H, _E, _D = 32, 2048, 64, 1024
_J = 4
_GW = 2 * _D // _J
_DW = _D // _J


def _moe_body(x_ref, rw_ref, rb_ref, gu_ref, gub_ref, dp_ref, dpb_ref, p_ref,
              out_ref, scores_ref):
    e = pl.program_id(0)
    j = pl.program_id(1)

    @pl.when(jnp.logical_and(e == 0, j == 0))
    def _router():
        x = x_ref[...]
        logits = jax.lax.dot_general(
            x, rw_ref[...], (((1,), (1,)), ((), ())),
            preferred_element_type=jnp.float32) + rb_ref[...]
        lane = jax.lax.broadcasted_iota(jnp.int32, (_B, _E), 1)
        neg = jnp.float32(-1e30)
        work = logits
        sel = jnp.zeros((_B, _E), jnp.bool_)
        for _ in range(_TOP_K):
            m = jnp.max(work, axis=1, keepdims=True)
            cand = work == m
            idx = jnp.min(jnp.where(cand, lane, _E), axis=1, keepdims=True)
            pick = lane == idx
            sel = jnp.logical_or(sel, pick)
            work = jnp.where(pick, neg, work)
        mx = jnp.max(jnp.where(sel, logits, neg), axis=1, keepdims=True)
        ex = jnp.where(sel, jnp.exp(logits - mx), jnp.float32(0.0))
        scores_ref[...] = ex / jnp.sum(ex, axis=1, keepdims=True)

    x = x_ref[...]
    gu = jax.lax.dot_general(
        x, gu_ref[0], (((1,), (0,)), ((), ())),
        preferred_element_type=jnp.float32) + gub_ref[0]
    gate_v = jnp.minimum(gu, _LIMIT)
    glu = gate_v * jax.nn.sigmoid(gate_v * _ALPHA)
    up_v = jnp.clip(gu, -_LIMIT, _LIMIT) + 1.0
    up_shift = pltpu.roll(up_v, _GW - 1, 1)
    prod = (glu * up_shift).astype(jnp.bfloat16)
    act = jax.lax.dot_general(
        prod, p_ref[...], (((1,), (0,)), ((), ())),
        preferred_element_type=jnp.float32)
    y = jax.lax.dot_general(
        act, dp_ref[0], (((1,), (0,)), ((), ())),
        preferred_element_type=jnp.float32)
    y = y + dpb_ref[0] * jnp.where(j == 0, 1.0, 0.0).astype(jnp.float32)
    lane_e = jax.lax.broadcasted_iota(jnp.int32, (_B, _E), 1)
    s = jnp.sum(jnp.where(lane_e == e, scores_ref[...], jnp.float32(0.0)),
                axis=1, keepdims=True)
    contrib = y * s

    @pl.when(jnp.logical_and(e == 0, j == 0))
    def _init():
        out_ref[...] = contrib

    @pl.when(jnp.logical_or(e != 0, j != 0))
    def _acc():
        out_ref[...] += contrib


def kernel(hidden_states, router_weight, router_bias, gate_up_proj,
           gate_up_proj_bias, down_proj, down_proj_bias):
    batch = hidden_states.shape[0]
    x = hidden_states.reshape(-1, _H)
    rb = router_bias.reshape(1, _E)
    perm = (jax.lax.broadcasted_iota(jnp.int32, (_GW, _DW), 0)
            == 2 * jax.lax.broadcasted_iota(jnp.int32, (_GW, _DW), 1)
            ).astype(jnp.bfloat16)

    out = pl.pallas_call(
        _moe_body,
        grid=(_E, _J),
        in_specs=[
            pl.BlockSpec((_B, _H), lambda e, j: (0, 0)),
            pl.BlockSpec((_E, _H), lambda e, j: (0, 0)),
            pl.BlockSpec((1, _E), lambda e, j: (0, 0)),
            pl.BlockSpec((1, _H, _GW), lambda e, j: (e, 0, j)),
            pl.BlockSpec((1, 1, _GW), lambda e, j: (e, 0, j)),
            pl.BlockSpec((1, _DW, _H), lambda e, j: (e, j, 0)),
            pl.BlockSpec((1, 1, _H), lambda e, j: (e, 0, 0)),
            pl.BlockSpec((_GW, _DW), lambda e, j: (0, 0)),
        ],
        out_specs=pl.BlockSpec((_B, _H), lambda e, j: (0, 0)),
        out_shape=jax.ShapeDtypeStruct((_B, _H), jnp.float32),
        scratch_shapes=[pltpu.VMEM((_B, _E), jnp.float32)],
        compiler_params=pltpu.CompilerParams(
            dimension_semantics=("arbitrary", "arbitrary"),
            vmem_limit_bytes=100 * 1024 * 1024,
        ),
    )(x, router_weight, rb, gate_up_proj,
      gate_up_proj_bias.reshape(_E, 1, 2 * _D),
      down_proj, down_proj_bias.reshape(_E, 1, _H), perm)

    return out.reshape(batch, -1, _H)


# speedup vs baseline: 1.0825x; 1.0825x over previous
import jax
import jax.numpy as jnp
from jax.experimental import pallas as pl
from jax.experimental.pallas import tpu as pltpu

_ALPHA = 1.702
_LIMIT = 7.0
_TOP_K = 8
_B, _H, _E, _D = 32, 2048, 64, 1024
_HH = _H // 2


def _moe_body(x_ref, rw_ref, rb_ref, gu_ref, gub_ref, dp_ref, dpb_ref, p_ref,
              out_ref, scores_ref, acc_ref):
    e = pl.program_id(0)
    h = pl.program_id(1)

    @pl.when(jnp.logical_and(e == 0, h == 0))
    def _router():
        x = jnp.concatenate([x_ref[0], x_ref[1]], axis=1)
        logits = jax.lax.dot_general(
            x, rw_ref[...], (((1,), (1,)), ((), ())),
            preferred_element_type=jnp.float32) + rb_ref[...]
        lane = jax.lax.broadcasted_iota(jnp.int32, (_B, _E), 1)
        neg = jnp.float32(-1e30)
        work = logits
        sel = jnp.zeros((_B, _E), jnp.bool_)
        for _ in range(_TOP_K):
            m = jnp.max(work, axis=1, keepdims=True)
            cand = work == m
            idx = jnp.min(jnp.where(cand, lane, _E), axis=1, keepdims=True)
            pick = lane == idx
            sel = jnp.logical_or(sel, pick)
            work = jnp.where(pick, neg, work)
        mx = jnp.max(jnp.where(sel, logits, neg), axis=1, keepdims=True)
        ex = jnp.where(sel, jnp.exp(logits - mx), jnp.float32(0.0))
        scores_ref[...] = ex / jnp.sum(ex, axis=1, keepdims=True)

    part = jax.lax.dot_general(
        x_ref[h], gu_ref[0], (((1,), (0,)), ((), ())),
        preferred_element_type=jnp.float32)

    @pl.when(h == 0)
    def _stash():
        acc_ref[...] = part

    @pl.when(h == 1)
    def _finish():
        gu = acc_ref[...] + part + gub_ref[0]
        gate_v = jnp.minimum(gu, _LIMIT)
        glu = gate_v * jax.nn.sigmoid(gate_v * _ALPHA)
        up_v = jnp.clip(gu, -_LIMIT, _LIMIT) + 1.0
        up_shift = pltpu.roll(up_v, 2 * _D - 1, 1)
        prod = (glu * up_shift).astype(jnp.bfloat16)
        act = jax.lax.dot_general(
            prod, p_ref[...], (((1,), (0,)), ((), ())),
            preferred_element_type=jnp.float32)
        y = jax.lax.dot_general(
            act, dp_ref[0], (((1,), (0,)), ((), ())),
            preferred_element_type=jnp.float32) + dpb_ref[0]
        lane_e = jax.lax.broadcasted_iota(jnp.int32, (_B, _E), 1)
        s = jnp.sum(jnp.where(lane_e == e, scores_ref[...], jnp.float32(0.0)),
                    axis=1, keepdims=True)
        contrib = y * s

        @pl.when(e == 0)
        def _init():
            out_ref[...] = contrib

        @pl.when(e != 0)
        def _acc():
            out_ref[...] += contrib


def kernel(hidden_states, router_weight, router_bias, gate_up_proj,
           gate_up_proj_bias, down_proj, down_proj_bias):
    batch = hidden_states.shape[0]
    x = hidden_states.reshape(-1, _H)
    x2 = x.reshape(_B, 2, _HH).transpose(1, 0, 2)
    rb = router_bias.reshape(1, _E)
    perm = (jax.lax.broadcasted_iota(jnp.int32, (2 * _D, _D), 0)
            == 2 * jax.lax.broadcasted_iota(jnp.int32, (2 * _D, _D), 1)
            ).astype(jnp.bfloat16)

    out = pl.pallas_call(
        _moe_body,
        grid=(_E, 2),
        in_specs=[
            pl.BlockSpec((2, _B, _HH), lambda e, h: (0, 0, 0)),
            pl.BlockSpec((_E, _H), lambda e, h: (0, 0)),
            pl.BlockSpec((1, _E), lambda e, h: (0, 0)),
            pl.BlockSpec((1, _HH, 2 * _D), lambda e, h: (e, h, 0)),
            pl.BlockSpec((1, 1, 2 * _D), lambda e, h: (e, 0, 0)),
            pl.BlockSpec((1, _D, _H), lambda e, h: (e, 0, 0)),
            pl.BlockSpec((1, 1, _H), lambda e, h: (e, 0, 0)),
            pl.BlockSpec((2 * _D, _D), lambda e, h: (0, 0)),
        ],
        out_specs=pl.BlockSpec((_B, _H), lambda e, h: (0, 0)),
        out_shape=jax.ShapeDtypeStruct((_B, _H), jnp.float32),
        scratch_shapes=[pltpu.VMEM((_B, _E), jnp.float32),
                        pltpu.VMEM((_B, 2 * _D), jnp.float32)],
        compiler_params=pltpu.CompilerParams(
            dimension_semantics=("arbitrary", "arbitrary"),
            vmem_limit_bytes=100 * 1024 * 1024,
        ),
    )(x2, router_weight, rb, gate_up_proj,
      gate_up_proj_bias.reshape(_E, 1, 2 * _D),
      down_proj, down_proj_bias.reshape(_E, 1, _H), perm)

    return out.reshape(batch, -1, _H)


# speedup vs baseline: 1.0939x; 1.0105x over previous
import jax
import jax.numpy as jnp
from jax.experimental import pallas as pl
from jax.experimental.pallas import tpu as pltpu

_ALPHA = 1.702
_LIMIT = 7.0
_TOP_K = 8
_B, _H, _E, _D = 32, 2048, 64, 1024
_J = 2
_GW = 2 * _D // _J
_DW = _D // _J


def _moe_body(x_ref, rw_ref, rb_ref, gu_ref, gub_ref, dp_ref, dpb_ref, p_ref,
              out_ref, scores_ref):
    e = pl.program_id(0)
    j = pl.program_id(1)

    @pl.when(jnp.logical_and(e == 0, j == 0))
    def _router():
        x = x_ref[...]
        logits = jax.lax.dot_general(
            x, rw_ref[...], (((1,), (1,)), ((), ())),
            preferred_element_type=jnp.float32) + rb_ref[...]
        lane = jax.lax.broadcasted_iota(jnp.int32, (_B, _E), 1)
        neg = jnp.float32(-1e30)
        work = logits
        sel = jnp.zeros((_B, _E), jnp.bool_)
        for _ in range(_TOP_K):
            m = jnp.max(work, axis=1, keepdims=True)
            cand = work == m
            idx = jnp.min(jnp.where(cand, lane, _E), axis=1, keepdims=True)
            pick = lane == idx
            sel = jnp.logical_or(sel, pick)
            work = jnp.where(pick, neg, work)
        mx = jnp.max(jnp.where(sel, logits, neg), axis=1, keepdims=True)
        ex = jnp.where(sel, jnp.exp(logits - mx), jnp.float32(0.0))
        scores_ref[...] = ex / jnp.sum(ex, axis=1, keepdims=True)

    x = x_ref[...]
    gu = jax.lax.dot_general(
        x, gu_ref[0], (((1,), (0,)), ((), ())),
        preferred_element_type=jnp.float32) + gub_ref[0]
    gate_v = jnp.minimum(gu, _LIMIT)
    glu = gate_v * jax.nn.sigmoid(gate_v * _ALPHA)
    up_v = jnp.clip(gu, -_LIMIT, _LIMIT) + 1.0
    up_shift = pltpu.roll(up_v, _GW - 1, 1)
    prod = (glu * up_shift).astype(jnp.bfloat16)
    act = jax.lax.dot_general(
        prod, p_ref[...], (((1,), (0,)), ((), ())),
        preferred_element_type=jnp.float32)
    y = jax.lax.dot_general(
        act, dp_ref[0], (((1,), (0,)), ((), ())),
        preferred_element_type=jnp.float32)
    y = y + dpb_ref[0] * jnp.where(j == 0, 1.0, 0.0).astype(jnp.float32)
    lane_e = jax.lax.broadcasted_iota(jnp.int32, (_B, _E), 1)
    s = jnp.sum(jnp.where(lane_e == e, scores_ref[...], jnp.float32(0.0)),
                axis=1, keepdims=True)
    contrib = y * s

    @pl.when(jnp.logical_and(e == 0, j == 0))
    def _init():
        out_ref[...] = contrib

    @pl.when(jnp.logical_or(e != 0, j != 0))
    def _acc():
        out_ref[...] += contrib


def kernel(hidden_states, router_weight, router_bias, gate_up_proj,
           gate_up_proj_bias, down_proj, down_proj_bias):
    batch = hidden_states.shape[0]
    x = hidden_states.reshape(-1, _H)
    rb = router_bias.reshape(1, _E)
    perm = (jax.lax.broadcasted_iota(jnp.int32, (_GW, _DW), 0)
            == 2 * jax.lax.broadcasted_iota(jnp.int32, (_GW, _DW), 1)
            ).astype(jnp.bfloat16)

    out = pl.pallas_call(
        _moe_body,
        grid=(_E, _J),
        in_specs=[
            pl.BlockSpec((_B, _H), lambda e, j: (0, 0)),
            pl.BlockSpec((_E, _H), lambda e, j: (0, 0)),
            pl.BlockSpec((1, _E), lambda e, j: (0, 0)),
            pl.BlockSpec((1, _H, _GW), lambda e, j: (e, 0, j)),
            pl.BlockSpec((1, 1, _GW), lambda e, j: (e, 0, j)),
            pl.BlockSpec((1, _DW, _H), lambda e, j: (e, j, 0)),
            pl.BlockSpec((1, 1, _H), lambda e, j: (e, 0, 0)),
            pl.BlockSpec((_GW, _DW), lambda e, j: (0, 0)),
        ],
        out_specs=pl.BlockSpec((_B, _H), lambda e, j: (0, 0)),
        out_shape=jax.ShapeDtypeStruct((_B, _H), jnp.float32),
        scratch_shapes=[pltpu.VMEM((_B, _E), jnp.float32)],
        compiler_params=pltpu.CompilerParams(
            dimension_semantics=("arbitrary", "arbitrary"),
            vmem_limit_bytes=100 * 1024 * 1024,
        ),
    )(x, router_weight, rb, gate_up_proj,
      gate_up_proj_bias.reshape(_E, 1, 2 * _D),
      down_proj, down_proj_bias.reshape(_E, 1, _H), perm)

    return out.reshape(batch, -1, _H)
